# baseline (speedup 1.0000x reference)
import jax
import jax.numpy as jnp
from jax.experimental import pallas as pl
from jax.experimental.pallas import tpu as pltpu


_BM = 320


def _gcn_body(seq_ref, adj_ref, wt_ref, bias_ref, alpha_ref, out_ref, fts_ref):
    @pl.when(pl.program_id(0) == 0)
    def _():
        fts_ref[...] = jnp.dot(
            seq_ref[...], wt_ref[...], preferred_element_type=jnp.float32
        )

    o = jnp.dot(adj_ref[...], fts_ref[...], preferred_element_type=jnp.float32)
    o = o + bias_ref[...]
    alpha = alpha_ref[0]
    out_ref[...] = jnp.where(o >= 0, o, alpha * o)


def kernel(seq, adj, W, bias, alpha):
    b, n, d_in = seq.shape
    d_out = W.shape[0]
    seq2 = seq.reshape(n, d_in)
    adj2 = adj.reshape(n, n)
    wt = W.T
    bias2 = bias.reshape(1, d_out)
    alpha2 = jnp.reshape(alpha, (1,))

    grid = (pl.cdiv(n, _BM),)
    out = pl.pallas_call(
        _gcn_body,
        grid=grid,
        in_specs=[
            pl.BlockSpec((n, d_in), lambda i: (0, 0)),
            pl.BlockSpec((_BM, n), lambda i: (i, 0)),
            pl.BlockSpec((d_in, d_out), lambda i: (0, 0)),
            pl.BlockSpec((1, d_out), lambda i: (0, 0)),
            pl.BlockSpec(memory_space=pltpu.SMEM),
        ],
        out_specs=pl.BlockSpec((_BM, d_out), lambda i: (i, 0)),
        out_shape=jax.ShapeDtypeStruct((n, d_out), jnp.float32),
        scratch_shapes=[pltpu.VMEM((n, d_out), jnp.float32)],
    )(seq2, adj2, wt, bias2, alpha2)
    return out.reshape(b, n, d_out)


# speedup vs baseline: 1.0077x; 1.0077x over previous
import jax
import jax.numpy as jnp
from jax.experimental import pallas as pl
from jax.experimental.pallas import tpu as pltpu


_BM = 272


def _gcn_body(seq_ref, adj_ref, wt_ref, bias_ref, alpha_ref, out_ref, fts_ref):
    @pl.when(pl.program_id(0) == 0)
    def _():
        fts_ref[...] = jnp.dot(
            seq_ref[...], wt_ref[...], preferred_element_type=jnp.float32
        )

    o = jnp.dot(adj_ref[...], fts_ref[...], preferred_element_type=jnp.float32)
    o = o + bias_ref[...]
    alpha = alpha_ref[0]
    out_ref[...] = jnp.where(o >= 0, o, alpha * o)


def kernel(seq, adj, W, bias, alpha):
    b, n, d_in = seq.shape
    d_out = W.shape[0]
    seq2 = seq.reshape(n, d_in)
    adj2 = adj.reshape(n, n)
    wt = W.T
    bias2 = bias.reshape(1, d_out)
    alpha2 = jnp.reshape(alpha, (1,))

    grid = (pl.cdiv(n, _BM),)
    out = pl.pallas_call(
        _gcn_body,
        grid=grid,
        in_specs=[
            pl.BlockSpec((n, d_in), lambda i: (0, 0)),
            pl.BlockSpec((_BM, n), lambda i: (i, 0)),
            pl.BlockSpec((d_in, d_out), lambda i: (0, 0)),
            pl.BlockSpec((1, d_out), lambda i: (0, 0)),
            pl.BlockSpec(memory_space=pltpu.SMEM),
        ],
        out_specs=pl.BlockSpec((_BM, d_out), lambda i: (i, 0)),
        out_shape=jax.ShapeDtypeStruct((n, d_out), jnp.float32),
        scratch_shapes=[pltpu.VMEM((n, d_out), jnp.float32)],
    )(seq2, adj2, wt, bias2, alpha2)
    return out.reshape(b, n, d_out)


# speedup vs baseline: 1.0084x; 1.0007x over previous
import jax
import jax.numpy as jnp
from jax.experimental import pallas as pl
from jax.experimental.pallas import tpu as pltpu


_BM = 304


def _gcn_body(seq_ref, adj_ref, wt_ref, bias_ref, alpha_ref, out_ref, fts_ref):
    @pl.when(pl.program_id(0) == 0)
    def _():
        fts_ref[...] = jnp.dot(
            seq_ref[...], wt_ref[...], preferred_element_type=jnp.float32
        )

    o = jnp.dot(adj_ref[...], fts_ref[...], preferred_element_type=jnp.float32)
    o = o + bias_ref[...]
    alpha = alpha_ref[0]
    out_ref[...] = jnp.where(o >= 0, o, alpha * o)


def kernel(seq, adj, W, bias, alpha):
    b, n, d_in = seq.shape
    d_out = W.shape[0]
    seq2 = seq.reshape(n, d_in)
    adj2 = adj.reshape(n, n)
    wt = W.T
    bias2 = bias.reshape(1, d_out)
    alpha2 = jnp.reshape(alpha, (1,))

    grid = (pl.cdiv(n, _BM),)
    out = pl.pallas_call(
        _gcn_body,
        grid=grid,
        in_specs=[
            pl.BlockSpec((n, d_in), lambda i: (0, 0)),
            pl.BlockSpec((_BM, n), lambda i: (i, 0)),
            pl.BlockSpec((d_in, d_out), lambda i: (0, 0)),
            pl.BlockSpec((1, d_out), lambda i: (0, 0)),
            pl.BlockSpec(memory_space=pltpu.SMEM),
        ],
        out_specs=pl.BlockSpec((_BM, d_out), lambda i: (i, 0)),
        out_shape=jax.ShapeDtypeStruct((n, d_out), jnp.float32),
        scratch_shapes=[pltpu.VMEM((n, d_out), jnp.float32)],
    )(seq2, adj2, wt, bias2, alpha2)
    return out.reshape(b, n, d_out)
